# baseline (speedup 1.0000x reference)
import functools

import jax
import jax.numpy as jnp
from jax import lax
from jax.experimental import pallas as pl
from jax.experimental.pallas import tpu as pltpu
from jax.experimental.pallas import tpu_sc as plsc

_NC, _NS, _L = 2, 16, 16
_NW = _NC * _NS


def _sqdist_body(x1_ref, x2_ref, o_ref):
    a = x1_ref[...]
    b = x2_ref[...]
    g = lax.dot_general(a, b, (((1,), (1,)), ((), ())),
                        preferred_element_type=jnp.float32)
    na = jnp.sum(a * a, axis=1, keepdims=True)
    nb = jnp.sum(b * b, axis=1)
    s = (na - 2.0 * g) + nb[None, :]
    o_ref[...] = jnp.maximum(s, 0.0) + 1e-12


def _make_sc_sqrt(total, chunk=4096, unroll=8):
    per_w = total // _NW
    n_chunks = per_w // chunk
    mesh = plsc.VectorSubcoreMesh(core_axis_name="c", subcore_axis_name="s")

    @functools.partial(
        pl.kernel,
        out_type=jax.ShapeDtypeStruct((total,), jnp.float32),
        mesh=mesh,
        scratch_types=[
            pltpu.VMEM((chunk,), jnp.float32),
            pltpu.VMEM((chunk,), jnp.float32),
            pltpu.SemaphoreType.DMA,
            pltpu.SemaphoreType.DMA,
            pltpu.SemaphoreType.DMA,
            pltpu.SemaphoreType.DMA,
        ],
        compiler_params=pltpu.CompilerParams(
            needs_layout_passes=False, skip_device_barrier=True),
    )
    def sc_sqrt(s_hbm, out_hbm, b0, b1, si0, si1, so0, so1):
        wid = lax.axis_index("s") * _NC + lax.axis_index("c")
        base = wid * per_w
        bufs = (b0, b1)
        isems = (si0, si1)
        osems = (so0, so1)

        in_cp = {0: pltpu.async_copy(
            s_hbm.at[pl.ds(base, chunk)], b0, si0)}
        out_cp = {}
        for ci in range(n_chunks):
            buf = bufs[ci % 2]
            if ci + 1 < n_chunks:
                if ci - 1 >= 0:
                    out_cp[ci - 1].wait()
                in_cp[ci + 1] = pltpu.async_copy(
                    s_hbm.at[pl.ds(base + (ci + 1) * chunk, chunk)],
                    bufs[(ci + 1) % 2], isems[(ci + 1) % 2])
            in_cp[ci].wait()

            @plsc.parallel_loop(0, chunk // _L, unroll=unroll)
            def _vec(vi, buf=buf):
                s = buf[pl.ds(vi * _L, _L)]
                i = plsc.bitcast(s, jnp.int32)
                i = 0x5F3759DF - lax.shift_right_arithmetic(i, 1)
                r = plsc.bitcast(i, jnp.float32)
                h = 0.5 * s
                r = r * (1.5 - h * r * r)
                r = r * (1.5 - h * r * r)
                buf[pl.ds(vi * _L, _L)] = s * r

            out_cp[ci] = pltpu.async_copy(
                buf, out_hbm.at[pl.ds(base + ci * chunk, chunk)],
                osems[ci % 2])
        out_cp[n_chunks - 2].wait()
        out_cp[n_chunks - 1].wait()

    return sc_sqrt


_sc_sqrt = _make_sc_sqrt(1024 * 1024)


def kernel(x1, x2):
    n1 = x1.shape[0]
    n2 = x2.shape[0]
    s = pl.pallas_call(
        _sqdist_body,
        out_shape=jax.ShapeDtypeStruct((n1, n2), jnp.float32),
    )(x1, x2)
    d = _sc_sqrt(s.reshape(n1 * n2))
    return d.reshape(n1, n2)


# speedup vs baseline: 1.2428x; 1.2428x over previous
import functools

import jax
import jax.numpy as jnp
from jax import lax
from jax.experimental import pallas as pl
from jax.experimental.pallas import tpu as pltpu
from jax.experimental.pallas import tpu_sc as plsc

_NC, _NS, _L = 2, 16, 16
_NW = _NC * _NS


def _sqdist_body(x1_ref, x2_ref, o_ref):
    a = x1_ref[...]
    b = x2_ref[...]
    g = lax.dot_general(a, b, (((1,), (1,)), ((), ())),
                        preferred_element_type=jnp.float32)
    na = jnp.sum(a * a, axis=1, keepdims=True)
    nb = jnp.sum(b * b, axis=1)
    s = (na - 2.0 * g) + nb[None, :]
    o_ref[...] = jnp.maximum(s, 0.0) + 1e-12


def _dist_body(x1_ref, x2_ref, o_ref):
    a = x1_ref[...]
    b = x2_ref[...]
    g = lax.dot_general(a, b, (((1,), (1,)), ((), ())),
                        preferred_element_type=jnp.float32)
    na = jnp.sum(a * a, axis=1, keepdims=True)
    nb = jnp.sum(b * b, axis=1)
    s = (na - 2.0 * g) + nb[None, :]
    o_ref[...] = jnp.sqrt(jnp.maximum(s, 0.0) + 1e-12)


def _make_sc_sqrt(total, chunk=4096, unroll=8):
    per_w = total // _NW
    n_chunks = per_w // chunk
    mesh = plsc.VectorSubcoreMesh(core_axis_name="c", subcore_axis_name="s")

    @functools.partial(
        pl.kernel,
        out_type=jax.ShapeDtypeStruct((total,), jnp.float32),
        mesh=mesh,
        scratch_types=[
            pltpu.VMEM((chunk,), jnp.float32),
            pltpu.VMEM((chunk,), jnp.float32),
            pltpu.SemaphoreType.DMA,
            pltpu.SemaphoreType.DMA,
            pltpu.SemaphoreType.DMA,
            pltpu.SemaphoreType.DMA,
        ],
        compiler_params=pltpu.CompilerParams(needs_layout_passes=False),
    )
    def sc_sqrt(s_hbm, out_hbm, b0, b1, si0, si1, so0, so1):
        wid = lax.axis_index("s") * _NC + lax.axis_index("c")
        base = wid * per_w
        bufs = (b0, b1)
        isems = (si0, si1)
        osems = (so0, so1)

        in_cp = {0: pltpu.async_copy(
            s_hbm.at[pl.ds(base, chunk)], b0, si0)}
        out_cp = {}
        for ci in range(n_chunks):
            buf = bufs[ci % 2]
            if ci + 1 < n_chunks:
                if ci - 1 >= 0:
                    out_cp[ci - 1].wait()
                in_cp[ci + 1] = pltpu.async_copy(
                    s_hbm.at[pl.ds(base + (ci + 1) * chunk, chunk)],
                    bufs[(ci + 1) % 2], isems[(ci + 1) % 2])
            in_cp[ci].wait()

            @plsc.parallel_loop(0, chunk // _L, unroll=unroll)
            def _vec(vi, buf=buf):
                s = buf[pl.ds(vi * _L, _L)]
                i = plsc.bitcast(s, jnp.int32)
                i = 0x5F3759DF - lax.shift_right_arithmetic(i, 1)
                r = plsc.bitcast(i, jnp.float32)
                h = 0.5 * s
                r = r * (1.5 - h * r * r)
                r = r * (1.5 - h * r * r)
                buf[pl.ds(vi * _L, _L)] = s * r

            out_cp[ci] = pltpu.async_copy(
                buf, out_hbm.at[pl.ds(base + ci * chunk, chunk)],
                osems[ci % 2])
        out_cp[n_chunks - 2].wait()
        out_cp[n_chunks - 1].wait()

    return sc_sqrt


_SC_ROWS = 256
_sc_sqrt = _make_sc_sqrt(_SC_ROWS * 1024)


def kernel(x1, x2):
    n1 = x1.shape[0]
    n2 = x2.shape[0]
    tc_rows = n1 - _SC_ROWS
    s_sc = pl.pallas_call(
        _sqdist_body,
        out_shape=jax.ShapeDtypeStruct((_SC_ROWS, n2), jnp.float32),
    )(x1[tc_rows:], x2)
    d_sc = _sc_sqrt(s_sc.reshape(_SC_ROWS * n2))
    d_tc = pl.pallas_call(
        _dist_body,
        out_shape=jax.ShapeDtypeStruct((tc_rows, n2), jnp.float32),
    )(x1[:tc_rows], x2)
    return jnp.concatenate([d_tc, d_sc.reshape(_SC_ROWS, n2)], axis=0)


# speedup vs baseline: 1.2693x; 1.0213x over previous
import functools

import jax
import jax.numpy as jnp
from jax import lax
from jax.experimental import pallas as pl
from jax.experimental.pallas import tpu as pltpu
from jax.experimental.pallas import tpu_sc as plsc

_NC, _NS, _L = 2, 16, 16
_NW = _NC * _NS


def _sqdist_body(x1_ref, x2_ref, o_ref):
    a = x1_ref[...]
    b = x2_ref[...]
    g = lax.dot_general(a, b, (((1,), (1,)), ((), ())),
                        preferred_element_type=jnp.float32)
    na = jnp.sum(a * a, axis=1, keepdims=True)
    nb = jnp.sum(b * b, axis=1)
    s = (na - 2.0 * g) + nb[None, :]
    o_ref[...] = jnp.maximum(s, 0.0) + 1e-12


def _dist_split_body(x1_ref, x2_ref, d_ref, s_ref):
    tc_rows = d_ref.shape[0]
    a = x1_ref[...]
    b = x2_ref[...]
    g = lax.dot_general(a, b, (((1,), (1,)), ((), ())),
                        preferred_element_type=jnp.float32)
    na = jnp.sum(a * a, axis=1, keepdims=True)
    nb = jnp.sum(b * b, axis=1)
    s = (na - 2.0 * g) + nb[None, :]
    s = jnp.maximum(s, 0.0) + 1e-12
    d_ref[...] = jnp.sqrt(s[:tc_rows])
    s_ref[...] = s[tc_rows:]


def _make_sc_sqrt(total, chunk=4096, unroll=8):
    per_w = total // _NW
    n_chunks = per_w // chunk
    mesh = plsc.VectorSubcoreMesh(core_axis_name="c", subcore_axis_name="s")

    @functools.partial(
        pl.kernel,
        out_type=jax.ShapeDtypeStruct((total,), jnp.float32),
        mesh=mesh,
        scratch_types=[
            pltpu.VMEM((chunk,), jnp.float32),
            pltpu.VMEM((chunk,), jnp.float32),
            pltpu.SemaphoreType.DMA,
            pltpu.SemaphoreType.DMA,
            pltpu.SemaphoreType.DMA,
            pltpu.SemaphoreType.DMA,
        ],
        compiler_params=pltpu.CompilerParams(needs_layout_passes=False),
    )
    def sc_sqrt(s_hbm, out_hbm, b0, b1, si0, si1, so0, so1):
        wid = lax.axis_index("s") * _NC + lax.axis_index("c")
        base = wid * per_w
        bufs = (b0, b1)
        isems = (si0, si1)
        osems = (so0, so1)

        in_cp = {0: pltpu.async_copy(
            s_hbm.at[pl.ds(base, chunk)], b0, si0)}
        out_cp = {}
        for ci in range(n_chunks):
            buf = bufs[ci % 2]
            if ci + 1 < n_chunks:
                if ci - 1 >= 0:
                    out_cp[ci - 1].wait()
                in_cp[ci + 1] = pltpu.async_copy(
                    s_hbm.at[pl.ds(base + (ci + 1) * chunk, chunk)],
                    bufs[(ci + 1) % 2], isems[(ci + 1) % 2])
            in_cp[ci].wait()

            @plsc.parallel_loop(0, chunk // _L, unroll=unroll)
            def _vec(vi, buf=buf):
                s = buf[pl.ds(vi * _L, _L)]
                i = plsc.bitcast(s, jnp.int32)
                i = 0x5F3759DF - lax.shift_right_arithmetic(i, 1)
                r = plsc.bitcast(i, jnp.float32)
                h = 0.5 * s
                r = r * (1.5 - h * r * r)
                r = r * (1.5 - h * r * r)
                buf[pl.ds(vi * _L, _L)] = s * r

            out_cp[ci] = pltpu.async_copy(
                buf, out_hbm.at[pl.ds(base + ci * chunk, chunk)],
                osems[ci % 2])
        out_cp[n_chunks - 2].wait()
        out_cp[n_chunks - 1].wait()

    return sc_sqrt


_SC_ROWS = 256
_sc_sqrt = _make_sc_sqrt(_SC_ROWS * 1024)


def kernel(x1, x2):
    n1 = x1.shape[0]
    n2 = x2.shape[0]
    tc_rows = n1 - _SC_ROWS
    d_tc, s_sc = pl.pallas_call(
        _dist_split_body,
        out_shape=[
            jax.ShapeDtypeStruct((tc_rows, n2), jnp.float32),
            jax.ShapeDtypeStruct((_SC_ROWS, n2), jnp.float32),
        ],
    )(x1, x2)
    d_sc = _sc_sqrt(s_sc.reshape(_SC_ROWS * n2))
    return jnp.concatenate([d_tc, d_sc.reshape(_SC_ROWS, n2)], axis=0)


# speedup vs baseline: 1.4049x; 1.1068x over previous
import functools

import jax
import jax.numpy as jnp
from jax import lax
from jax.experimental import pallas as pl
from jax.experimental.pallas import tpu as pltpu
from jax.experimental.pallas import tpu_sc as plsc

_NC, _NS, _L = 2, 16, 16
_NW = _NC * _NS


def _sqdist_body(x1_ref, x2_ref, o_ref):
    a = x1_ref[...]
    b = x2_ref[...]
    g = lax.dot_general(a, b, (((1,), (1,)), ((), ())),
                        preferred_element_type=jnp.float32)
    na = jnp.sum(a * a, axis=1, keepdims=True)
    nb = jnp.sum(b * b, axis=1)
    s = (na - 2.0 * g) + nb[None, :]
    o_ref[...] = jnp.maximum(s, 0.0) + 1e-12


def _dist_pad_body(x1_ref, x2_ref, o_ref):
    tc_rows = x1_ref.shape[0]
    a = x1_ref[...]
    b = x2_ref[...]
    g = lax.dot_general(a, b, (((1,), (1,)), ((), ())),
                        preferred_element_type=jnp.float32)
    na = jnp.sum(a * a, axis=1, keepdims=True)
    nb = jnp.sum(b * b, axis=1)
    s = (na - 2.0 * g) + nb[None, :]
    o_ref[0:tc_rows, :] = jnp.sqrt(jnp.maximum(s, 0.0) + 1e-12)
    o_ref[tc_rows:, :] = jnp.zeros(
        (o_ref.shape[0] - tc_rows, o_ref.shape[1]), jnp.float32)


def _make_sc_sqrt(total, chunk=4096, unroll=8):
    per_w = total // _NW
    n_chunks = per_w // chunk
    mesh = plsc.VectorSubcoreMesh(core_axis_name="c", subcore_axis_name="s")

    @functools.partial(
        pl.kernel,
        out_type=jax.ShapeDtypeStruct((total,), jnp.float32),
        mesh=mesh,
        scratch_types=[
            pltpu.VMEM((chunk,), jnp.float32),
            pltpu.VMEM((chunk,), jnp.float32),
            pltpu.SemaphoreType.DMA,
            pltpu.SemaphoreType.DMA,
            pltpu.SemaphoreType.DMA,
            pltpu.SemaphoreType.DMA,
        ],
        compiler_params=pltpu.CompilerParams(needs_layout_passes=False),
    )
    def sc_sqrt(s_hbm, out_hbm, b0, b1, si0, si1, so0, so1):
        wid = lax.axis_index("s") * _NC + lax.axis_index("c")
        base = wid * per_w
        bufs = (b0, b1)
        isems = (si0, si1)
        osems = (so0, so1)

        in_cp = {0: pltpu.async_copy(
            s_hbm.at[pl.ds(base, chunk)], b0, si0)}
        out_cp = {}
        for ci in range(n_chunks):
            buf = bufs[ci % 2]
            if ci + 1 < n_chunks:
                if ci - 1 >= 0:
                    out_cp[ci - 1].wait()
                in_cp[ci + 1] = pltpu.async_copy(
                    s_hbm.at[pl.ds(base + (ci + 1) * chunk, chunk)],
                    bufs[(ci + 1) % 2], isems[(ci + 1) % 2])
            in_cp[ci].wait()

            @plsc.parallel_loop(0, chunk // _L, unroll=unroll)
            def _vec(vi, buf=buf):
                s = buf[pl.ds(vi * _L, _L)]
                i = plsc.bitcast(s, jnp.int32)
                i = 0x5F3759DF - lax.shift_right_arithmetic(i, 1)
                r = plsc.bitcast(i, jnp.float32)
                h = 0.5 * s
                r = r * (1.5 - h * r * r)
                r = r * (1.5 - h * r * r)
                buf[pl.ds(vi * _L, _L)] = s * r

            out_cp[ci] = pltpu.async_copy(
                buf, out_hbm.at[pl.ds(base + ci * chunk, chunk)],
                osems[ci % 2])
        out_cp[n_chunks - 2].wait()
        out_cp[n_chunks - 1].wait()

    return sc_sqrt


_SC_ROWS = 256
_sc_sqrt = _make_sc_sqrt(_SC_ROWS * 1024)


def kernel(x1, x2):
    n1 = x1.shape[0]
    n2 = x2.shape[0]
    tc_rows = n1 - _SC_ROWS
    s_sc = pl.pallas_call(
        _sqdist_body,
        grid=(1,),
        in_specs=[
            pl.BlockSpec((_SC_ROWS, x1.shape[1]),
                         lambda i: (tc_rows // _SC_ROWS, 0)),
            pl.BlockSpec((n2, x2.shape[1]), lambda i: (0, 0)),
        ],
        out_specs=pl.BlockSpec((_SC_ROWS, n2), lambda i: (0, 0)),
        out_shape=jax.ShapeDtypeStruct((_SC_ROWS, n2), jnp.float32),
    )(x1, x2)
    d_sc = _sc_sqrt(s_sc.reshape(_SC_ROWS * n2))
    d_full = pl.pallas_call(
        _dist_pad_body,
        grid=(1,),
        in_specs=[
            pl.BlockSpec((tc_rows, x1.shape[1]), lambda i: (0, 0)),
            pl.BlockSpec((n2, x2.shape[1]), lambda i: (0, 0)),
        ],
        out_specs=pl.BlockSpec((n1, n2), lambda i: (0, 0)),
        out_shape=jax.ShapeDtypeStruct((n1, n2), jnp.float32),
    )(x1, x2)
    return lax.dynamic_update_slice(
        d_full, d_sc.reshape(_SC_ROWS, n2), (tc_rows, 0))


# speedup vs baseline: 1.4726x; 1.0481x over previous
import functools

import jax
import jax.numpy as jnp
from jax import lax
from jax.experimental import pallas as pl
from jax.experimental.pallas import tpu as pltpu
from jax.experimental.pallas import tpu_sc as plsc

_NC, _NS, _L = 2, 16, 16
_NW = _NC * _NS


def _sqdist_body(x1_ref, x2_ref, o_ref):
    a = x1_ref[...]
    b = x2_ref[...]
    g = lax.dot_general(a, b, (((1,), (1,)), ((), ())),
                        preferred_element_type=jnp.float32)
    na = jnp.sum(a * a, axis=1, keepdims=True)
    nb = jnp.sum(b * b, axis=1)
    s = (na - 2.0 * g) + nb[None, :]
    o_ref[...] = jnp.maximum(s, 0.0) + 1e-12


def _dist_pad_body(x1_ref, x2_ref, o_ref):
    tc_rows = x1_ref.shape[0]
    a = x1_ref[...]
    b = x2_ref[...]
    g = lax.dot_general(a, b, (((1,), (1,)), ((), ())),
                        preferred_element_type=jnp.float32)
    na = jnp.sum(a * a, axis=1, keepdims=True)
    nb = jnp.sum(b * b, axis=1)
    s = (na - 2.0 * g) + nb[None, :]
    o_ref[0:tc_rows, :] = jnp.sqrt(jnp.maximum(s, 0.0) + 1e-12)
    o_ref[tc_rows:, :] = jnp.zeros(
        (o_ref.shape[0] - tc_rows, o_ref.shape[1]), jnp.float32)


def _make_sc_sqrt(total, chunk=4096, unroll=8):
    per_w = total // _NW
    n_chunks = per_w // chunk
    mesh = plsc.VectorSubcoreMesh(core_axis_name="c", subcore_axis_name="s")

    @functools.partial(
        pl.kernel,
        out_type=jax.ShapeDtypeStruct((total,), jnp.float32),
        mesh=mesh,
        scratch_types=[
            pltpu.VMEM((chunk,), jnp.float32),
            pltpu.VMEM((chunk,), jnp.float32),
            pltpu.SemaphoreType.DMA,
            pltpu.SemaphoreType.DMA,
            pltpu.SemaphoreType.DMA,
            pltpu.SemaphoreType.DMA,
        ],
        compiler_params=pltpu.CompilerParams(needs_layout_passes=False),
    )
    def sc_sqrt(s_hbm, out_hbm, b0, b1, si0, si1, so0, so1):
        wid = lax.axis_index("s") * _NC + lax.axis_index("c")
        base = wid * per_w
        bufs = (b0, b1)
        isems = (si0, si1)
        osems = (so0, so1)

        in_cp = {0: pltpu.async_copy(
            s_hbm.at[pl.ds(base, chunk)], b0, si0)}
        out_cp = {}
        for ci in range(n_chunks):
            buf = bufs[ci % 2]
            if ci + 1 < n_chunks:
                if ci - 1 >= 0:
                    out_cp[ci - 1].wait()
                in_cp[ci + 1] = pltpu.async_copy(
                    s_hbm.at[pl.ds(base + (ci + 1) * chunk, chunk)],
                    bufs[(ci + 1) % 2], isems[(ci + 1) % 2])
            in_cp[ci].wait()

            @plsc.parallel_loop(0, chunk // _L, unroll=unroll)
            def _vec(vi, buf=buf):
                s = buf[pl.ds(vi * _L, _L)]
                i = plsc.bitcast(s, jnp.int32)
                i = 0x5F3759DF - lax.shift_right_arithmetic(i, 1)
                r = plsc.bitcast(i, jnp.float32)
                h = 0.5 * s
                r = r * (1.5 - h * r * r)
                r = r * (1.5 - h * r * r)
                buf[pl.ds(vi * _L, _L)] = s * r

            out_cp[ci] = pltpu.async_copy(
                buf, out_hbm.at[pl.ds(base + ci * chunk, chunk)],
                osems[ci % 2])
        if n_chunks >= 2:
            out_cp[n_chunks - 2].wait()
        out_cp[n_chunks - 1].wait()

    return sc_sqrt


_SC_ROWS = 128
_sc_sqrt = _make_sc_sqrt(_SC_ROWS * 1024)


def kernel(x1, x2):
    n1 = x1.shape[0]
    n2 = x2.shape[0]
    tc_rows = n1 - _SC_ROWS
    s_sc = pl.pallas_call(
        _sqdist_body,
        grid=(1,),
        in_specs=[
            pl.BlockSpec((_SC_ROWS, x1.shape[1]),
                         lambda i: (tc_rows // _SC_ROWS, 0)),
            pl.BlockSpec((n2, x2.shape[1]), lambda i: (0, 0)),
        ],
        out_specs=pl.BlockSpec((_SC_ROWS, n2), lambda i: (0, 0)),
        out_shape=jax.ShapeDtypeStruct((_SC_ROWS, n2), jnp.float32),
    )(x1, x2)
    d_sc = _sc_sqrt(s_sc.reshape(_SC_ROWS * n2))
    d_full = pl.pallas_call(
        _dist_pad_body,
        grid=(1,),
        in_specs=[
            pl.BlockSpec((tc_rows, x1.shape[1]), lambda i: (0, 0)),
            pl.BlockSpec((n2, x2.shape[1]), lambda i: (0, 0)),
        ],
        out_specs=pl.BlockSpec((n1, n2), lambda i: (0, 0)),
        out_shape=jax.ShapeDtypeStruct((n1, n2), jnp.float32),
    )(x1, x2)
    return lax.dynamic_update_slice(
        d_full, d_sc.reshape(_SC_ROWS, n2), (tc_rows, 0))


# speedup vs baseline: 1.4930x; 1.0139x over previous
import functools

import jax
import jax.numpy as jnp
from jax import lax
from jax.experimental import pallas as pl
from jax.experimental.pallas import tpu as pltpu
from jax.experimental.pallas import tpu_sc as plsc

_NC, _NS, _L = 2, 16, 16
_NW = _NC * _NS


def _sqdist_body(x1_ref, x2_ref, o_ref):
    a = x1_ref[...]
    b = x2_ref[...]
    g = lax.dot_general(a, b, (((1,), (1,)), ((), ())),
                        preferred_element_type=jnp.float32)
    na = jnp.sum(a * a, axis=1, keepdims=True)
    nb = jnp.sum(b * b, axis=1)
    s = (na - 2.0 * g) + nb[None, :]
    o_ref[...] = jnp.maximum(s, 0.0) + 1e-12


def _dist_pad_body(x1_ref, x2_ref, o_ref):
    tc_rows = x1_ref.shape[0]
    a = x1_ref[...]
    b = x2_ref[...]
    g = lax.dot_general(a, b, (((1,), (1,)), ((), ())),
                        preferred_element_type=jnp.float32)
    na = jnp.sum(a * a, axis=1, keepdims=True)
    nb = jnp.sum(b * b, axis=1)
    s = (na - 2.0 * g) + nb[None, :]
    o_ref[0:tc_rows, :] = jnp.sqrt(jnp.maximum(s, 0.0) + 1e-12)
    o_ref[tc_rows:, :] = jnp.zeros(
        (o_ref.shape[0] - tc_rows, o_ref.shape[1]), jnp.float32)


def _make_sc_sqrt(total, chunk=4096, unroll=8):
    per_w = total // _NW
    chunk = min(chunk, per_w)
    n_chunks = per_w // chunk
    mesh = plsc.VectorSubcoreMesh(core_axis_name="c", subcore_axis_name="s")

    @functools.partial(
        pl.kernel,
        out_type=jax.ShapeDtypeStruct((total,), jnp.float32),
        mesh=mesh,
        scratch_types=[
            pltpu.VMEM((chunk,), jnp.float32),
            pltpu.VMEM((chunk,), jnp.float32),
            pltpu.SemaphoreType.DMA,
            pltpu.SemaphoreType.DMA,
            pltpu.SemaphoreType.DMA,
            pltpu.SemaphoreType.DMA,
        ],
        compiler_params=pltpu.CompilerParams(needs_layout_passes=False),
    )
    def sc_sqrt(s_hbm, out_hbm, b0, b1, si0, si1, so0, so1):
        wid = lax.axis_index("s") * _NC + lax.axis_index("c")
        base = wid * per_w
        bufs = (b0, b1)
        isems = (si0, si1)
        osems = (so0, so1)

        in_cp = {0: pltpu.async_copy(
            s_hbm.at[pl.ds(base, chunk)], b0, si0)}
        out_cp = {}
        for ci in range(n_chunks):
            buf = bufs[ci % 2]
            if ci + 1 < n_chunks:
                if ci - 1 >= 0:
                    out_cp[ci - 1].wait()
                in_cp[ci + 1] = pltpu.async_copy(
                    s_hbm.at[pl.ds(base + (ci + 1) * chunk, chunk)],
                    bufs[(ci + 1) % 2], isems[(ci + 1) % 2])
            in_cp[ci].wait()

            @plsc.parallel_loop(0, chunk // _L, unroll=unroll)
            def _vec(vi, buf=buf):
                s = buf[pl.ds(vi * _L, _L)]
                i = plsc.bitcast(s, jnp.int32)
                i = 0x5F3759DF - lax.shift_right_arithmetic(i, 1)
                r = plsc.bitcast(i, jnp.float32)
                h = 0.5 * s
                r = r * (1.5 - h * r * r)
                r = r * (1.5 - h * r * r)
                buf[pl.ds(vi * _L, _L)] = s * r

            out_cp[ci] = pltpu.async_copy(
                buf, out_hbm.at[pl.ds(base + ci * chunk, chunk)],
                osems[ci % 2])
        if n_chunks >= 2:
            out_cp[n_chunks - 2].wait()
        out_cp[n_chunks - 1].wait()

    return sc_sqrt


_SC_ROWS = 64
_sc_sqrt = _make_sc_sqrt(_SC_ROWS * 1024)


def kernel(x1, x2):
    n1 = x1.shape[0]
    n2 = x2.shape[0]
    tc_rows = n1 - _SC_ROWS
    s_sc = pl.pallas_call(
        _sqdist_body,
        grid=(1,),
        in_specs=[
            pl.BlockSpec((_SC_ROWS, x1.shape[1]),
                         lambda i: (tc_rows // _SC_ROWS, 0)),
            pl.BlockSpec((n2, x2.shape[1]), lambda i: (0, 0)),
        ],
        out_specs=pl.BlockSpec((_SC_ROWS, n2), lambda i: (0, 0)),
        out_shape=jax.ShapeDtypeStruct((_SC_ROWS, n2), jnp.float32),
    )(x1, x2)
    d_sc = _sc_sqrt(s_sc.reshape(_SC_ROWS * n2))
    d_full = pl.pallas_call(
        _dist_pad_body,
        grid=(1,),
        in_specs=[
            pl.BlockSpec((tc_rows, x1.shape[1]), lambda i: (0, 0)),
            pl.BlockSpec((n2, x2.shape[1]), lambda i: (0, 0)),
        ],
        out_specs=pl.BlockSpec((n1, n2), lambda i: (0, 0)),
        out_shape=jax.ShapeDtypeStruct((n1, n2), jnp.float32),
    )(x1, x2)
    return lax.dynamic_update_slice(
        d_full, d_sc.reshape(_SC_ROWS, n2), (tc_rows, 0))
